# Initial kernel scaffold; baseline (speedup 1.0000x reference)
import functools

import jax
import jax.numpy as jnp
from jax import lax
from jax.experimental import pallas as pl
from jax.experimental.pallas import tpu as pltpu
from jax.experimental.pallas import tpu_sc as plsc

N = 10000
E = 320000
C = 128
R = 8
NSEG = R * N
SEG_ROWS = 5120
SEG_PAD = SEG_ROWS * 16

NW = 32
EPW = E // NW
SUB = 5
ESUB = EPW // SUB
CHUNK = 80
NCHUNK = ESUB // CHUNK
ROWS_PT = N // 16
CNT_RPT = SEG_ROWS // 16


def _mesh():
    return plsc.VectorSubcoreMesh(core_axis_name="c", subcore_axis_name="s")


def _sc_count(typ, dst):
    @functools.partial(
        pl.kernel,
        mesh=_mesh(),
        out_type=jax.ShapeDtypeStruct((2, SEG_ROWS, 16), jnp.float32),
        scratch_types=[
            pltpu.VMEM((EPW,), jnp.int32),
            pltpu.VMEM((EPW,), jnp.int32),
            pltpu.VMEM((CHUNK, 16), jnp.float32),
            pltpu.VMEM((CHUNK,), jnp.int32),
            pltpu.VMEM((CHUNK,), jnp.int32),
            pltpu.VMEM((80, 16), jnp.float32),
            pltpu.VMEM_SHARED((SEG_ROWS, 16), jnp.float32),
        ],
    )
    def k(typ_hbm, dst_hbm, out_hbm, typ_v, dst_v, oneh, rowb, colb, zb,
          cnt_sh):
        cc = lax.axis_index("c")
        ss = lax.axis_index("s")
        wid = ss * 2 + cc
        zeros = jnp.zeros((16,), jnp.float32)
        ones = jnp.full((16,), 1.0, jnp.float32)
        iota = lax.iota(jnp.int32, 16)

        def _zrow(i, _):
            zb[i, :] = zeros
            return 0
        lax.fori_loop(0, 80, _zrow, 0)
        for k2 in range(CNT_RPT // 80):
            pltpu.sync_copy(zb, cnt_sh.at[pl.ds(ss * CNT_RPT + k2 * 80, 80)])

        def _zoh(i, _):
            oneh[i, :] = zeros
            return 0
        lax.fori_loop(0, CHUNK, _zoh, 0)
        pltpu.sync_copy(typ_hbm.at[pl.ds(wid * EPW, EPW)], typ_v)
        pltpu.sync_copy(dst_hbm.at[pl.ds(wid * EPW, EPW)], dst_v)
        plsc.subcore_barrier()

        def chunk(i, _):
            b = i * CHUNK
            for g in range(CHUNK // 16):
                sl = pl.ds(b + g * 16, 16)
                seg = typ_v[sl] * N + dst_v[sl]
                rowb[pl.ds(g * 16, 16)] = lax.shift_right_logical(seg, 4)
                col = lax.bitwise_and(seg, 15)
                colb[pl.ds(g * 16, 16)] = col
                plsc.store_scatter(oneh, [iota + g * 16, col], ones)
            pltpu.sync_copy(oneh, cnt_sh.at[rowb], add=True)
            for g in range(CHUNK // 16):
                plsc.store_scatter(
                    oneh, [iota + g * 16, colb[pl.ds(g * 16, 16)]], zeros)
            return 0
        lax.fori_loop(0, EPW // CHUNK, chunk, 0)
        plsc.subcore_barrier()
        for k2 in range(CNT_RPT // 80):
            base = ss * CNT_RPT + k2 * 80
            pltpu.sync_copy(cnt_sh.at[pl.ds(base, 80)], zb)
            pltpu.sync_copy(zb, out_hbm.at[cc].at[pl.ds(base, 80)])

    return k(typ, dst)


def _sc_scatter(y2d, cnt_inv, src, dst, typ):
    @functools.partial(
        pl.kernel,
        mesh=_mesh(),
        out_type=jax.ShapeDtypeStruct((2, N, C), jnp.float32),
        scratch_types=[
            pltpu.VMEM((SEG_PAD,), jnp.float32),
            pltpu.VMEM((ESUB,), jnp.int32),
            pltpu.VMEM((ESUB,), jnp.int32),
            pltpu.VMEM((ESUB,), jnp.int32),
            pltpu.VMEM((CHUNK, C), jnp.float32),
            pltpu.VMEM((CHUNK,), jnp.int32),
            pltpu.VMEM((CHUNK,), jnp.int32),
            pltpu.VMEM((CHUNK,), jnp.float32),
            pltpu.VMEM((125, C), jnp.float32),
            pltpu.VMEM_SHARED((N, C), jnp.float32),
            pltpu.SemaphoreType.DMA,
        ],
    )
    def k(y_hbm, ci_hbm, src_hbm, dst_hbm, typ_hbm, out_hbm,
          ci_v, src_v, dst_v, typ_v, rows, gix, dstb, scl, zb, acc, sem):
        cc = lax.axis_index("c")
        ss = lax.axis_index("s")
        wid = ss * 2 + cc
        zeros = jnp.zeros((16,), jnp.float32)

        def _zrow(i, _):
            for j in range(C // 16):
                zb[i, pl.ds(j * 16, 16)] = zeros
            return 0
        lax.fori_loop(0, 125, _zrow, 0)
        for k2 in range(ROWS_PT // 125):
            pltpu.sync_copy(zb, acc.at[pl.ds(ss * ROWS_PT + k2 * 125, 125)])
        pltpu.sync_copy(ci_hbm, ci_v)
        plsc.subcore_barrier()

        for sb in range(SUB):
            ebase = wid * EPW + sb * ESUB
            pltpu.sync_copy(src_hbm.at[pl.ds(ebase, ESUB)], src_v)
            pltpu.sync_copy(dst_hbm.at[pl.ds(ebase, ESUB)], dst_v)
            pltpu.sync_copy(typ_hbm.at[pl.ds(ebase, ESUB)], typ_v)

            def chunk(i, _):
                b = i * CHUNK
                for g in range(CHUNK // 16):
                    sl = pl.ds(b + g * 16, 16)
                    t = typ_v[sl]
                    d = dst_v[sl]
                    gix[pl.ds(g * 16, 16)] = t * N + src_v[sl]
                    dstb[pl.ds(g * 16, 16)] = d
                    scl[pl.ds(g * 16, 16)] = plsc.load_gather(
                        ci_v, [t * N + d])
                pltpu.async_copy(y_hbm.at[gix], rows, sem).wait()
                for j in range(CHUNK):
                    sv = scl[j]
                    for q in range(C // 16):
                        sl = pl.ds(q * 16, 16)
                        rows[j, sl] = rows[j, sl] * sv
                pltpu.sync_copy(rows, acc.at[dstb], add=True)
                return 0
            lax.fori_loop(0, NCHUNK, chunk, 0)

        plsc.subcore_barrier()
        for k2 in range(ROWS_PT // 125):
            base = ss * ROWS_PT + k2 * 125
            pltpu.sync_copy(acc.at[pl.ds(base, 125)], zb)
            pltpu.sync_copy(zb, out_hbm.at[cc].at[pl.ds(base, 125)])

    return k(y2d, cnt_inv, src, dst, typ)


def _tc_project(x, weight):
    BN = 1000

    def body(x_ref, w_ref, o_ref):
        o_ref[0] = jnp.dot(x_ref[...], w_ref[0],
                           preferred_element_type=jnp.float32)

    y = pl.pallas_call(
        body,
        grid=(R, N // BN),
        in_specs=[
            pl.BlockSpec((BN, C), lambda r, n: (n, 0)),
            pl.BlockSpec((1, C, C), lambda r, n: (r, 0, 0)),
        ],
        out_specs=pl.BlockSpec((1, BN, C), lambda r, n: (r, n, 0)),
        out_shape=jax.ShapeDtypeStruct((R, N, C), jnp.float32),
    )(x, weight)
    return y.reshape(R * N, C)


def _tc_cnt_inv(cnt_part):
    def body(p_ref, o_ref):
        o_ref[...] = 1.0 / jnp.maximum(p_ref[0] + p_ref[1], 1.0)

    p = cnt_part.reshape(2, SEG_PAD // 128, 128)
    ci = pl.pallas_call(
        body,
        out_shape=jax.ShapeDtypeStruct((SEG_PAD // 128, 128), jnp.float32),
    )(p)
    return ci.reshape(SEG_PAD)


def _tc_final(part, x, root, bias):
    BN = 1000

    def body(p_ref, x_ref, r_ref, b_ref, o_ref):
        o_ref[...] = (p_ref[0] + p_ref[1]
                      + jnp.dot(x_ref[...], r_ref[...],
                                preferred_element_type=jnp.float32)
                      + b_ref[...])

    return pl.pallas_call(
        body,
        grid=(N // BN,),
        in_specs=[
            pl.BlockSpec((2, BN, C), lambda n: (0, n, 0)),
            pl.BlockSpec((BN, C), lambda n: (n, 0)),
            pl.BlockSpec((C, C), lambda n: (0, 0)),
            pl.BlockSpec((1, C), lambda n: (0, 0)),
        ],
        out_specs=pl.BlockSpec((BN, C), lambda n: (n, 0)),
        out_shape=jax.ShapeDtypeStruct((N, C), jnp.float32),
    )(part, x, root, bias.reshape(1, C))


def kernel(x, edge_index, edge_type, weight, root, bias):
    src = edge_index[0].astype(jnp.int32)
    dst = edge_index[1].astype(jnp.int32)
    typ = edge_type.astype(jnp.int32)

    cnt_part = _sc_count(typ, dst)
    cnt_inv = _tc_cnt_inv(cnt_part)
    y2d = _tc_project(x, weight)
    part = _sc_scatter(y2d, cnt_inv, src, dst, typ)
    return _tc_final(part, x, root, bias)


# speedup vs baseline: 6.7946x; 6.7946x over previous
import functools

import jax
import jax.numpy as jnp
from jax import lax
from jax.experimental import pallas as pl
from jax.experimental.pallas import tpu as pltpu
from jax.experimental.pallas import tpu_sc as plsc

N = 10000
E = 320000
C = 128
R = 8
NSEG = R * N
SEG_ROWS = 5120
SEG_PAD = SEG_ROWS * 16

NW = 32
EPW = E // NW
SUB = 5
ESUB = EPW // SUB
CHUNK = 80
NCHUNK = ESUB // CHUNK
ROWS_PT = N // 16
CNT_RPT = SEG_ROWS // 16


def _mesh():
    return plsc.VectorSubcoreMesh(core_axis_name="c", subcore_axis_name="s")


def _sc_count(typ, dst):
    ZCH = 1280
    CPT = SEG_PAD // 16

    @functools.partial(
        pl.kernel,
        mesh=_mesh(),
        out_type=jax.ShapeDtypeStruct((2, SEG_PAD), jnp.float32),
        scratch_types=[
            pltpu.VMEM((EPW,), jnp.int32),
            pltpu.VMEM((EPW,), jnp.int32),
            pltpu.VMEM((CHUNK,), jnp.int32),
            pltpu.VMEM((CHUNK,), jnp.float32),
            pltpu.VMEM((ZCH,), jnp.float32),
            pltpu.VMEM_SHARED((SEG_PAD,), jnp.float32),
        ],
    )
    def k(typ_hbm, dst_hbm, out_hbm, typ_v, dst_v, segb, onesb, zb, cnt_sh):
        cc = lax.axis_index("c")
        ss = lax.axis_index("s")
        wid = ss * 2 + cc
        zeros = jnp.zeros((16,), jnp.float32)
        ones = jnp.full((16,), 1.0, jnp.float32)

        def _z(i, _):
            zb[pl.ds(i * 16, 16)] = zeros
            return 0
        lax.fori_loop(0, ZCH // 16, _z, 0)
        for g in range(CHUNK // 16):
            onesb[pl.ds(g * 16, 16)] = ones
        for k2 in range(CPT // ZCH):
            pltpu.sync_copy(zb, cnt_sh.at[pl.ds(ss * CPT + k2 * ZCH, ZCH)])
        pltpu.sync_copy(typ_hbm.at[pl.ds(wid * EPW, EPW)], typ_v)
        pltpu.sync_copy(dst_hbm.at[pl.ds(wid * EPW, EPW)], dst_v)
        plsc.subcore_barrier()

        def chunk(i, _):
            b = i * CHUNK
            for g in range(CHUNK // 16):
                sl = pl.ds(b + g * 16, 16)
                segb[pl.ds(g * 16, 16)] = typ_v[sl] * N + dst_v[sl]
            pltpu.sync_copy(onesb, cnt_sh.at[segb], add=True)
            return 0
        lax.fori_loop(0, EPW // CHUNK, chunk, 0)
        plsc.subcore_barrier()
        for k2 in range(CPT // ZCH):
            base = ss * CPT + k2 * ZCH
            pltpu.sync_copy(cnt_sh.at[pl.ds(base, ZCH)], zb)
            pltpu.sync_copy(zb, out_hbm.at[cc].at[pl.ds(base, ZCH)])

    return k(typ, dst)


def _sc_scatter(y2d, cnt_inv, src, dst, typ):
    @functools.partial(
        pl.kernel,
        mesh=_mesh(),
        out_type=jax.ShapeDtypeStruct((2, N, C), jnp.float32),
        scratch_types=[
            pltpu.VMEM((ESUB,), jnp.int32),
            pltpu.VMEM((ESUB,), jnp.int32),
            pltpu.VMEM((ESUB,), jnp.int32),
            pltpu.VMEM((CHUNK, C), jnp.float32),
            pltpu.VMEM((CHUNK,), jnp.int32),
            pltpu.VMEM((CHUNK,), jnp.int32),
            pltpu.VMEM((CHUNK,), jnp.int32),
            pltpu.VMEM((CHUNK,), jnp.float32),
            pltpu.VMEM((80, C), jnp.float32),
            pltpu.VMEM_SHARED((N, C), jnp.float32),
            pltpu.SemaphoreType.DMA,
            pltpu.SemaphoreType.DMA,
        ],
        compiler_params=pltpu.CompilerParams(needs_layout_passes=False),
    )
    def k(y_hbm, ci_hbm, src_hbm, dst_hbm, typ_hbm, out_hbm,
          src_v, dst_v, typ_v, rows, gix, dstb, csegb, scl, zb, acc,
          sem, sem2):
        cc = lax.axis_index("c")
        ss = lax.axis_index("s")
        wid = ss * 2 + cc
        zeros = jnp.zeros((16,), jnp.float32)

        def _zrow(i, _):
            for j in range(C // 16):
                zb[i, pl.ds(j * 16, 16)] = zeros
            return 0
        lax.fori_loop(0, 80, _zrow, 0)

        def _zacc(ch, _):
            @pl.when(lax.bitwise_and(ch, 15) == ss)
            def _():
                pltpu.sync_copy(zb, acc.at[pl.ds(ch * 80, 80)])
            return 0
        lax.fori_loop(0, N // 80, _zacc, 0)
        plsc.subcore_barrier()

        for sb in range(SUB):
            ebase = wid * EPW + sb * ESUB
            pltpu.sync_copy(src_hbm.at[pl.ds(ebase, ESUB)], src_v)
            pltpu.sync_copy(dst_hbm.at[pl.ds(ebase, ESUB)], dst_v)
            pltpu.sync_copy(typ_hbm.at[pl.ds(ebase, ESUB)], typ_v)

            def chunk(i, _):
                b = i * CHUNK
                for g in range(CHUNK // 16):
                    sl = pl.ds(b + g * 16, 16)
                    t = typ_v[sl]
                    d = dst_v[sl]
                    gix[pl.ds(g * 16, 16)] = t * N + src_v[sl]
                    dstb[pl.ds(g * 16, 16)] = d
                    csegb[pl.ds(g * 16, 16)] = t * N + d
                cp1 = pltpu.async_copy(y_hbm.at[gix], rows, sem)
                cp2 = pltpu.async_copy(ci_hbm.at[csegb], scl, sem2)
                cp1.wait()
                cp2.wait()
                for g in range(CHUNK // 16):
                    svec = scl[pl.ds(g * 16, 16)]
                    for l in range(16):
                        j = g * 16 + l
                        sv = svec[l]
                        for q in range(C // 16):
                            sl = pl.ds(q * 16, 16)
                            rows[j, sl] = rows[j, sl] * sv
                pltpu.sync_copy(rows, acc.at[dstb], add=True)
                return 0
            lax.fori_loop(0, NCHUNK, chunk, 0)

        plsc.subcore_barrier()

        def _wb(ch, _):
            @pl.when(lax.bitwise_and(ch, 15) == ss)
            def _():
                pltpu.sync_copy(acc.at[pl.ds(ch * 80, 80)], zb)
                pltpu.sync_copy(zb, out_hbm.at[cc].at[pl.ds(ch * 80, 80)])
            return 0
        lax.fori_loop(0, N // 80, _wb, 0)

    return k(y2d, cnt_inv, src, dst, typ)


def _tc_project(x, weight):
    BN = 1000

    def body(x_ref, w_ref, o_ref):
        o_ref[0] = jnp.dot(x_ref[...], w_ref[0],
                           preferred_element_type=jnp.float32)

    y = pl.pallas_call(
        body,
        grid=(R, N // BN),
        in_specs=[
            pl.BlockSpec((BN, C), lambda r, n: (n, 0)),
            pl.BlockSpec((1, C, C), lambda r, n: (r, 0, 0)),
        ],
        out_specs=pl.BlockSpec((1, BN, C), lambda r, n: (r, n, 0)),
        out_shape=jax.ShapeDtypeStruct((R, N, C), jnp.float32),
    )(x, weight)
    return y.reshape(R * N, C)


def _tc_cnt_inv(cnt_part):
    def body(p_ref, o_ref):
        o_ref[...] = 1.0 / jnp.maximum(p_ref[0] + p_ref[1], 1.0)

    p = cnt_part.reshape(2, SEG_PAD // 128, 128)
    return pl.pallas_call(
        body,
        out_shape=jax.ShapeDtypeStruct((SEG_PAD // 128, 128), jnp.float32),
    )(p)


def _tc_final(part, x, root, bias):
    BN = 1000

    def body(p_ref, x_ref, r_ref, b_ref, o_ref):
        o_ref[...] = (p_ref[0] + p_ref[1]
                      + jnp.dot(x_ref[...], r_ref[...],
                                preferred_element_type=jnp.float32)
                      + b_ref[...])

    return pl.pallas_call(
        body,
        grid=(N // BN,),
        in_specs=[
            pl.BlockSpec((2, BN, C), lambda n: (0, n, 0)),
            pl.BlockSpec((BN, C), lambda n: (n, 0)),
            pl.BlockSpec((C, C), lambda n: (0, 0)),
            pl.BlockSpec((1, C), lambda n: (0, 0)),
        ],
        out_specs=pl.BlockSpec((BN, C), lambda n: (n, 0)),
        out_shape=jax.ShapeDtypeStruct((N, C), jnp.float32),
    )(part, x, root, bias.reshape(1, C))


def kernel(x, edge_index, edge_type, weight, root, bias):
    src = edge_index[0].astype(jnp.int32)
    dst = edge_index[1].astype(jnp.int32)
    typ = edge_type.astype(jnp.int32)

    cnt_part = _sc_count(typ, dst)
    cnt_inv = _tc_cnt_inv(cnt_part).reshape(SEG_PAD)
    y2d = _tc_project(x, weight)
    part = _sc_scatter(y2d, cnt_inv, src, dst, typ)
    return _tc_final(part, x, root, bias)


# speedup vs baseline: 7.9211x; 1.1658x over previous
import functools

import jax
import jax.numpy as jnp
from jax import lax
from jax.experimental import pallas as pl
from jax.experimental.pallas import tpu as pltpu
from jax.experimental.pallas import tpu_sc as plsc

N = 10000
E = 320000
C = 128
R = 8
NSEG = R * N
SEG_ROWS = 5120
SEG_PAD = SEG_ROWS * 16

NW = 32
EPW = E // NW
SUB = 5
ESUB = EPW // SUB
CHUNK = 80
NCHUNK = ESUB // CHUNK
ROWS_PT = N // 16
CNT_RPT = SEG_ROWS // 16


def _mesh():
    return plsc.VectorSubcoreMesh(core_axis_name="c", subcore_axis_name="s")


def _sc_count(typ, dst):
    ZCH = 1280
    CPT = SEG_PAD // 16

    @functools.partial(
        pl.kernel,
        mesh=_mesh(),
        out_type=jax.ShapeDtypeStruct((2, SEG_PAD), jnp.float32),
        scratch_types=[
            pltpu.VMEM((EPW,), jnp.int32),
            pltpu.VMEM((EPW,), jnp.int32),
            pltpu.VMEM((CHUNK,), jnp.int32),
            pltpu.VMEM((CHUNK,), jnp.float32),
            pltpu.VMEM((ZCH,), jnp.float32),
            pltpu.VMEM_SHARED((SEG_PAD,), jnp.float32),
        ],
    )
    def k(typ_hbm, dst_hbm, out_hbm, typ_v, dst_v, segb, onesb, zb, cnt_sh):
        cc = lax.axis_index("c")
        ss = lax.axis_index("s")
        wid = ss * 2 + cc
        zeros = jnp.zeros((16,), jnp.float32)
        ones = jnp.full((16,), 1.0, jnp.float32)

        def _z(i, _):
            zb[pl.ds(i * 16, 16)] = zeros
            return 0
        lax.fori_loop(0, ZCH // 16, _z, 0)
        for g in range(CHUNK // 16):
            onesb[pl.ds(g * 16, 16)] = ones
        for k2 in range(CPT // ZCH):
            pltpu.sync_copy(zb, cnt_sh.at[pl.ds(ss * CPT + k2 * ZCH, ZCH)])
        pltpu.sync_copy(typ_hbm.at[pl.ds(wid * EPW, EPW)], typ_v)
        pltpu.sync_copy(dst_hbm.at[pl.ds(wid * EPW, EPW)], dst_v)
        plsc.subcore_barrier()

        def chunk(i, _):
            b = i * CHUNK
            for g in range(CHUNK // 16):
                sl = pl.ds(b + g * 16, 16)
                segb[pl.ds(g * 16, 16)] = typ_v[sl] * N + dst_v[sl]
            pltpu.sync_copy(onesb, cnt_sh.at[segb], add=True)
            return 0
        lax.fori_loop(0, EPW // CHUNK, chunk, 0)
        plsc.subcore_barrier()
        for k2 in range(CPT // ZCH):
            base = ss * CPT + k2 * ZCH
            pltpu.sync_copy(cnt_sh.at[pl.ds(base, ZCH)], zb)
            pltpu.sync_copy(zb, out_hbm.at[cc].at[pl.ds(base, ZCH)])

    return k(typ, dst)


NBUF = 3


def _sc_scatter(y2d, cnt_inv, src, dst, typ):
    @functools.partial(
        pl.kernel,
        mesh=_mesh(),
        out_type=jax.ShapeDtypeStruct((2, N, C), jnp.float32),
        scratch_types=[
            pltpu.VMEM((ESUB,), jnp.int32),
            pltpu.VMEM((ESUB,), jnp.int32),
            pltpu.VMEM((ESUB,), jnp.int32),
            pltpu.VMEM((NBUF, CHUNK, C), jnp.float32),
            pltpu.VMEM((NBUF, CHUNK), jnp.int32),
            pltpu.VMEM((NBUF, CHUNK), jnp.int32),
            pltpu.VMEM((NBUF, CHUNK), jnp.int32),
            pltpu.VMEM((NBUF, CHUNK), jnp.float32),
            pltpu.VMEM((80, C), jnp.float32),
            pltpu.VMEM_SHARED((N, C), jnp.float32),
        ] + [pltpu.SemaphoreType.DMA] * 9,
        compiler_params=pltpu.CompilerParams(needs_layout_passes=False),
    )
    def k(y_hbm, ci_hbm, src_hbm, dst_hbm, typ_hbm, out_hbm,
          src_v, dst_v, typ_v, rows3, gix3, dstb3, cseg3, scl3, zb, acc,
          sg0, sg1, sg2, sc0, sc1, sc2, sa0, sa1, sa2):
        SG = [sg0, sg1, sg2]
        SCL = [sc0, sc1, sc2]
        SA = [sa0, sa1, sa2]
        cc = lax.axis_index("c")
        ss = lax.axis_index("s")
        wid = ss * 2 + cc
        zeros = jnp.zeros((16,), jnp.float32)

        def idx_fetch(c, slot):
            b = c * CHUNK
            for g in range(CHUNK // 16):
                sl = pl.ds(b + g * 16, 16)
                t = typ_v[sl]
                d = dst_v[sl]
                gix3[slot, pl.ds(g * 16, 16)] = t * N + src_v[sl]
                dstb3[slot, pl.ds(g * 16, 16)] = d
                cseg3[slot, pl.ds(g * 16, 16)] = t * N + d
            pltpu.async_copy(y_hbm.at[gix3.at[slot]], rows3.at[slot],
                             SG[slot])
            pltpu.async_copy(ci_hbm.at[cseg3.at[slot]], scl3.at[slot],
                             SCL[slot])

        def wait_fetch(slot):
            pltpu.make_async_copy(y_hbm.at[gix3.at[slot]], rows3.at[slot],
                                  SG[slot]).wait()
            pltpu.make_async_copy(ci_hbm.at[cseg3.at[slot]], scl3.at[slot],
                                  SCL[slot]).wait()

        def start_add(slot):
            pltpu.async_copy(rows3.at[slot], acc.at[dstb3.at[slot]],
                             SA[slot], add=True)

        def wait_add(slot):
            pltpu.make_async_copy(rows3.at[slot], acc.at[dstb3.at[slot]],
                                  SA[slot]).wait()

        def scale(slot):
            for g in range(CHUNK // 16):
                svec = scl3[slot, pl.ds(g * 16, 16)]
                for l in range(16):
                    j = g * 16 + l
                    sv = svec[l]
                    for q in range(C // 16):
                        sl = pl.ds(q * 16, 16)
                        rows3[slot, j, sl] = rows3[slot, j, sl] * sv

        def _zrow(i, _):
            for j in range(C // 16):
                zb[i, pl.ds(j * 16, 16)] = zeros
            return 0
        lax.fori_loop(0, 80, _zrow, 0)

        def _zacc(ch, _):
            @pl.when(lax.bitwise_and(ch, 15) == ss)
            def _():
                pltpu.sync_copy(zb, acc.at[pl.ds(ch * 80, 80)])
            return 0
        lax.fori_loop(0, N // 80, _zacc, 0)
        plsc.subcore_barrier()

        def emit_triple(c0, first):
            for kk in range(NBUF):
                nslot = (kk + 1) % NBUF
                if not (first and kk < NBUF - 1):
                    wait_add(nslot)
                idx_fetch(c0 + kk + 1, nslot)
                wait_fetch(kk)
                scale(kk)
                start_add(kk)

        def subblock(sb, _):
            ebase = wid * EPW + sb * ESUB
            pltpu.sync_copy(src_hbm.at[pl.ds(ebase, ESUB)], src_v)
            pltpu.sync_copy(dst_hbm.at[pl.ds(ebase, ESUB)], dst_v)
            pltpu.sync_copy(typ_hbm.at[pl.ds(ebase, ESUB)], typ_v)
            idx_fetch(0, 0)
            emit_triple(0, True)

            def triple(i, _):
                emit_triple(i * NBUF, False)
                return 0
            lax.fori_loop(1, NCHUNK // NBUF, triple, 0)
            wait_fetch(0)
            scale(0)
            start_add(0)
            for s2 in range(NBUF):
                wait_add(s2)
            return 0
        lax.fori_loop(0, SUB, subblock, 0)

        plsc.subcore_barrier()

        def _wb(ch, _):
            @pl.when(lax.bitwise_and(ch, 15) == ss)
            def _():
                pltpu.sync_copy(acc.at[pl.ds(ch * 80, 80)], zb)
                pltpu.sync_copy(zb, out_hbm.at[cc].at[pl.ds(ch * 80, 80)])
            return 0
        lax.fori_loop(0, N // 80, _wb, 0)

    return k(y2d, cnt_inv, src, dst, typ)


def _tc_project(x, weight):
    BN = 1000

    def body(x_ref, w_ref, o_ref):
        o_ref[0] = jnp.dot(x_ref[...], w_ref[0],
                           preferred_element_type=jnp.float32)

    y = pl.pallas_call(
        body,
        grid=(R, N // BN),
        in_specs=[
            pl.BlockSpec((BN, C), lambda r, n: (n, 0)),
            pl.BlockSpec((1, C, C), lambda r, n: (r, 0, 0)),
        ],
        out_specs=pl.BlockSpec((1, BN, C), lambda r, n: (r, n, 0)),
        out_shape=jax.ShapeDtypeStruct((R, N, C), jnp.float32),
    )(x, weight)
    return y.reshape(R * N, C)


def _tc_cnt_inv(cnt_part):
    def body(p_ref, o_ref):
        o_ref[...] = 1.0 / jnp.maximum(p_ref[0] + p_ref[1], 1.0)

    p = cnt_part.reshape(2, SEG_PAD // 128, 128)
    return pl.pallas_call(
        body,
        out_shape=jax.ShapeDtypeStruct((SEG_PAD // 128, 128), jnp.float32),
    )(p)


def _tc_final(part, x, root, bias):
    BN = 1000

    def body(p_ref, x_ref, r_ref, b_ref, o_ref):
        o_ref[...] = (p_ref[0] + p_ref[1]
                      + jnp.dot(x_ref[...], r_ref[...],
                                preferred_element_type=jnp.float32)
                      + b_ref[...])

    return pl.pallas_call(
        body,
        grid=(N // BN,),
        in_specs=[
            pl.BlockSpec((2, BN, C), lambda n: (0, n, 0)),
            pl.BlockSpec((BN, C), lambda n: (n, 0)),
            pl.BlockSpec((C, C), lambda n: (0, 0)),
            pl.BlockSpec((1, C), lambda n: (0, 0)),
        ],
        out_specs=pl.BlockSpec((BN, C), lambda n: (n, 0)),
        out_shape=jax.ShapeDtypeStruct((N, C), jnp.float32),
    )(part, x, root, bias.reshape(1, C))


def kernel(x, edge_index, edge_type, weight, root, bias):
    src = edge_index[0].astype(jnp.int32)
    dst = edge_index[1].astype(jnp.int32)
    typ = edge_type.astype(jnp.int32)

    cnt_part = _sc_count(typ, dst)
    cnt_inv = _tc_cnt_inv(cnt_part).reshape(SEG_PAD)
    y2d = _tc_project(x, weight)
    part = _sc_scatter(y2d, cnt_inv, src, dst, typ)
    return _tc_final(part, x, root, bias)


# speedup vs baseline: 8.5258x; 1.0763x over previous
import functools

import jax
import jax.numpy as jnp
from jax import lax
from jax.experimental import pallas as pl
from jax.experimental.pallas import tpu as pltpu
from jax.experimental.pallas import tpu_sc as plsc

N = 10000
E = 320000
C = 128
R = 8
NSEG = R * N
SEG_ROWS = 5120
SEG_PAD = SEG_ROWS * 16

NW = 32
EPW = E // NW
SUB = 5
ESUB = EPW // SUB
CHUNK = 80
NCHUNK = ESUB // CHUNK
ROWS_PT = N // 16
CNT_RPT = SEG_ROWS // 16


def _mesh():
    return plsc.VectorSubcoreMesh(core_axis_name="c", subcore_axis_name="s")


NBUF = 3
CIST = 1280
CPT = SEG_PAD // 16


def _sc_main(y2d, ei, et):
    @functools.partial(
        pl.kernel,
        mesh=_mesh(),
        out_type=jax.ShapeDtypeStruct((2, N, C), jnp.float32),
        scratch_types=[
            pltpu.VMEM((ESUB,), jnp.int32),
            pltpu.VMEM((ESUB,), jnp.int32),
            pltpu.VMEM((ESUB,), jnp.int32),
            pltpu.VMEM((NBUF, CHUNK, C), jnp.float32),
            pltpu.VMEM((NBUF, CHUNK), jnp.int32),
            pltpu.VMEM((NBUF, CHUNK), jnp.int32),
            pltpu.VMEM((NBUF, CHUNK), jnp.int32),
            pltpu.VMEM((NBUF, CHUNK), jnp.float32),
            pltpu.VMEM((CHUNK,), jnp.float32),
            pltpu.VMEM((40, C), jnp.float32),
            pltpu.VMEM((CIST,), jnp.float32),
            pltpu.VMEM_SHARED((N, C), jnp.float32),
            pltpu.VMEM_SHARED((SEG_PAD,), jnp.float32),
        ] + [pltpu.SemaphoreType.DMA] * 9,
        compiler_params=pltpu.CompilerParams(needs_layout_passes=False),
    )
    def k(y_hbm, ei_hbm, et_hbm, out_hbm,
          src_v, dst_v, typ_v, rows3, gix3, dstb3, cseg3, scl3, onesb, zb,
          cist, acc, cnt_sh, sg0, sg1, sg2, sc0, sc1, sc2, sa0, sa1, sa2):
        SG = [sg0, sg1, sg2]
        SCL = [sc0, sc1, sc2]
        SA = [sa0, sa1, sa2]
        cc = lax.axis_index("c")
        ss = lax.axis_index("s")
        wid = ss * 2 + cc
        zeros = jnp.zeros((16,), jnp.float32)
        ones = jnp.full((16,), 1.0, jnp.float32)

        def _zrow(i, _):
            for j in range(C // 16):
                zb[i, pl.ds(j * 16, 16)] = zeros
            return 0
        lax.fori_loop(0, 40, _zrow, 0)

        def _zci(i, _):
            cist[pl.ds(i * 16, 16)] = zeros
            return 0
        lax.fori_loop(0, CIST // 16, _zci, 0)
        for g in range(CHUNK // 16):
            onesb[pl.ds(g * 16, 16)] = ones
        for k2 in range(CPT // CIST):
            pltpu.sync_copy(cist, cnt_sh.at[pl.ds(ss * CPT + k2 * CIST,
                                                  CIST)])

        def _zacc(ch, _):
            @pl.when(lax.bitwise_and(ch, 15) == ss)
            def _():
                pltpu.sync_copy(zb, acc.at[pl.ds(ch * 40, 40)])
            return 0
        lax.fori_loop(0, N // 40, _zacc, 0)
        plsc.subcore_barrier()

        CB = [(cseg3, 0), (cseg3, 1), (cseg3, 2), (gix3, 0)]
        CS = [sg0, sg1, sc0, sc1]

        def cidx(c, slot):
            ref, row = CB[slot]
            b = c * CHUNK
            for g in range(CHUNK // 16):
                sl = pl.ds(b + g * 16, 16)
                ref[row, pl.ds(g * 16, 16)] = typ_v[sl] * N + dst_v[sl]

        def cstart(slot):
            ref, row = CB[slot]
            pltpu.async_copy(onesb, cnt_sh.at[ref.at[row]], CS[slot],
                             add=True)

        def cwait(slot):
            ref, row = CB[slot]
            pltpu.make_async_copy(onesb, cnt_sh.at[ref.at[row]],
                                  CS[slot]).wait()

        def csub(sb, _):
            ebase = ss * (E // 16) + sb * ESUB
            pltpu.sync_copy(et_hbm.at[pl.ds(ebase, ESUB)], typ_v)
            pltpu.sync_copy(ei_hbm.at[pl.ds(E + ebase, ESUB)], dst_v)
            cidx(0, 0)
            cstart(0)

            def cquad(c0, first):
                for kk in range(4):
                    tgt = c0 + kk + 1
                    slot = (kk + 1) % 4
                    if not (first and kk < 3):
                        cwait(slot)
                    cidx(tgt, slot)
                    cstart(slot)

            cquad(0, True)

            def quad(i, _):
                cquad(i * 4, False)
                return 0
            lax.fori_loop(1, (NCHUNK - 1) // 4, quad, 0)
            for s4 in range(4):
                cwait(s4)
            return 0
        lax.fori_loop(0, (E // 16) // ESUB, csub, 0)
        plsc.subcore_barrier()

        for r2 in range(CPT // CIST):
            base = ss * CPT + r2 * CIST
            pltpu.sync_copy(cnt_sh.at[pl.ds(base, CIST)], cist)

            def _inv(i, _):
                v = cist[pl.ds(i * 16, 16)]
                cist[pl.ds(i * 16, 16)] = 1.0 / jnp.maximum(v, 1.0)
                return 0
            lax.fori_loop(0, CIST // 16, _inv, 0)
            pltpu.sync_copy(cist, cnt_sh.at[pl.ds(base, CIST)])
        plsc.subcore_barrier()

        def idx_fetch(c, slot):
            b = c * CHUNK
            for g in range(CHUNK // 16):
                sl = pl.ds(b + g * 16, 16)
                t = typ_v[sl]
                d = dst_v[sl]
                gix3[slot, pl.ds(g * 16, 16)] = src_v[sl] * R + t
                dstb3[slot, pl.ds(g * 16, 16)] = d
                cseg3[slot, pl.ds(g * 16, 16)] = t * N + d
            pltpu.async_copy(y_hbm.at[gix3.at[slot]], rows3.at[slot],
                             SG[slot])
            pltpu.async_copy(cnt_sh.at[cseg3.at[slot]], scl3.at[slot],
                             SCL[slot])

        def wait_fetch(slot):
            pltpu.make_async_copy(y_hbm.at[gix3.at[slot]], rows3.at[slot],
                                  SG[slot]).wait()
            pltpu.make_async_copy(cnt_sh.at[cseg3.at[slot]], scl3.at[slot],
                                  SCL[slot]).wait()

        def start_add(slot):
            pltpu.async_copy(rows3.at[slot], acc.at[dstb3.at[slot]],
                             SA[slot], add=True)

        def wait_add(slot):
            pltpu.make_async_copy(rows3.at[slot], acc.at[dstb3.at[slot]],
                                  SA[slot]).wait()

        def scale(slot):
            for g in range(CHUNK // 16):
                svec = scl3[slot, pl.ds(g * 16, 16)]
                for l in range(16):
                    j = g * 16 + l
                    sv = svec[l]
                    for q in range(C // 16):
                        sl = pl.ds(q * 16, 16)
                        rows3[slot, j, sl] = rows3[slot, j, sl] * sv

        def emit_pair(c0, first):
            for kk in range(NBUF):
                nslot = (kk + 1) % NBUF
                if not (first and kk < NBUF - 1):
                    wait_add(nslot)
                idx_fetch(c0 + kk + 1, nslot)
                wait_fetch(kk)
                scale(kk)
                start_add(kk)

        def subblock(sb, _):
            ebase = wid * EPW + sb * ESUB
            pltpu.sync_copy(ei_hbm.at[pl.ds(ebase, ESUB)], src_v)
            pltpu.sync_copy(ei_hbm.at[pl.ds(E + ebase, ESUB)], dst_v)
            pltpu.sync_copy(et_hbm.at[pl.ds(ebase, ESUB)], typ_v)
            idx_fetch(0, 0)
            emit_pair(0, True)

            def pair3(i, _):
                emit_pair(i * NBUF, False)
                return 0
            lax.fori_loop(1, NCHUNK // NBUF, pair3, 0)
            wait_fetch(0)
            scale(0)
            start_add(0)
            for s2 in range(NBUF):
                wait_add(s2)
            return 0
        lax.fori_loop(0, SUB, subblock, 0)

        plsc.subcore_barrier()

        def _wb(ch, _):
            @pl.when(lax.bitwise_and(ch, 15) == ss)
            def _():
                pltpu.sync_copy(acc.at[pl.ds(ch * 40, 40)], zb)
                pltpu.sync_copy(zb, out_hbm.at[cc].at[pl.ds(ch * 40, 40)])
            return 0
        lax.fori_loop(0, N // 40, _wb, 0)

    return k(y2d, ei, et)


def _tc_project(x, weight):
    BN = 1000

    def body(x_ref, w_ref, o_ref):
        xb = x_ref[...]
        for r in range(R):
            o_ref[:, r, :] = jnp.dot(xb, w_ref[r],
                                     preferred_element_type=jnp.float32)

    return pl.pallas_call(
        body,
        grid=(N // BN,),
        in_specs=[
            pl.BlockSpec((BN, C), lambda n: (n, 0)),
            pl.BlockSpec((R, C, C), lambda n: (0, 0, 0)),
        ],
        out_specs=pl.BlockSpec((BN, R, C), lambda n: (n, 0, 0)),
        out_shape=jax.ShapeDtypeStruct((N, R, C), jnp.float32),
    )(x, weight)


def _tc_final(part, x, root, bias):
    BN = 1000

    def body(p_ref, x_ref, r_ref, b_ref, o_ref):
        o_ref[...] = (p_ref[0] + p_ref[1]
                      + jnp.dot(x_ref[...], r_ref[...],
                                preferred_element_type=jnp.float32)
                      + b_ref[...])

    return pl.pallas_call(
        body,
        grid=(N // BN,),
        in_specs=[
            pl.BlockSpec((2, BN, C), lambda n: (0, n, 0)),
            pl.BlockSpec((BN, C), lambda n: (n, 0)),
            pl.BlockSpec((C, C), lambda n: (0, 0)),
            pl.BlockSpec((1, C), lambda n: (0, 0)),
        ],
        out_specs=pl.BlockSpec((BN, C), lambda n: (n, 0)),
        out_shape=jax.ShapeDtypeStruct((N, C), jnp.float32),
    )(part, x, root, bias.reshape(1, C))


def kernel(x, edge_index, edge_type, weight, root, bias):
    ei = (edge_index if edge_index.dtype == jnp.int32
          else edge_index.astype(jnp.int32))
    et = (edge_type if edge_type.dtype == jnp.int32
          else edge_type.astype(jnp.int32))
    y = _tc_project(x, weight)
    part = _sc_main(y.reshape(N * R, C), ei.reshape(2 * E), et)
    return _tc_final(part, x, root, bias)
